# Initial kernel scaffold; baseline (speedup 1.0000x reference)
import jax
import jax.numpy as jnp
from jax.experimental import pallas as pl


def kernel(edge_index, edge_attr, batch, W, b):
    raise NotImplementedError("write your pallas kernel here")


# speedup vs baseline: 32.5422x; 32.5422x over previous
import jax
import jax.numpy as jnp
from jax import lax
from jax.experimental import pallas as pl
from jax.experimental.pallas import tpu as pltpu
from jax.experimental.pallas import tpu_sc as plsc

B = 256
N = 100000
E = 3200000
D = 16
DY = 64

NC = 2
NS = 16
NW = NC * NS
L = 16

EW = E // NW
CHUNK = 400
NCHUNKS = EW // CHUNK
NPACK = N // 4


def _sc_body(src_hbm, attr_hbm, batch_hbm,
             out_sum, out_sq, out_max, out_min, out_cnt,
             batch_v, idx_v, attr_v, seg_v,
             acc_sum, acc_sq, acc_max, acc_min, acc_cnt):
  c = lax.axis_index("c")
  s = lax.axis_index("s")
  wid = s * NC + c

  pltpu.sync_copy(batch_hbm, batch_v)

  zeros = jnp.zeros((L,), jnp.float32)
  ninf = jnp.full((L,), -jnp.inf, jnp.float32)
  pinf = jnp.full((L,), jnp.inf, jnp.float32)
  ones = jnp.ones((L,), jnp.float32)

  @pl.loop(0, B)
  def _init(r):
    row = pl.ds(r * L, L)
    acc_sum[row] = zeros
    acc_sq[row] = zeros
    acc_max[row] = ninf
    acc_min[row] = pinf
    acc_cnt[row] = zeros

  base0 = wid * EW

  @pl.loop(0, NCHUNKS)
  def _chunk(ci):
    base = base0 + ci * CHUNK
    pltpu.sync_copy(src_hbm.at[pl.ds(base, CHUNK)], idx_v)
    pltpu.sync_copy(attr_hbm.at[pl.ds(base * D, CHUNK * D)], attr_v)

    @pl.loop(0, CHUNK // L)
    def _group(g):
      iv = idx_v[pl.ds(g * L, L)]
      w = plsc.load_gather(batch_v, [lax.shift_right_logical(iv, 2)])
      sh = lax.shift_left(jnp.bitwise_and(iv, 3), 3)
      rows = lax.shift_left(
          jnp.bitwise_and(lax.shift_right_logical(w, sh), 255), 4)
      for j in range(L):
        row = pl.ds(rows[j], L)
        v = attr_v[pl.ds((g * L + j) * L, L)]
        plsc.addupdate(acc_sum.at[row], v)
        plsc.addupdate(acc_sq.at[row], v * v)
        plsc.addupdate(acc_cnt.at[row], ones)
        acc_max[row] = jnp.maximum(acc_max[row], v)
        acc_min[row] = jnp.minimum(acc_min[row], v)

  pltpu.sync_copy(acc_sum, out_sum.at[wid])
  pltpu.sync_copy(acc_sq, out_sq.at[wid])
  pltpu.sync_copy(acc_max, out_max.at[wid])
  pltpu.sync_copy(acc_min, out_min.at[wid])
  pltpu.sync_copy(acc_cnt, out_cnt.at[wid])


def _sc_partials(src, attr_flat, batch_packed):
  mesh = plsc.VectorSubcoreMesh(
      core_axis_name="c", subcore_axis_name="s",
      num_cores=NC, num_subcores=NS)
  f32 = jnp.float32
  part = jax.ShapeDtypeStruct((NW, B * L), f32)
  fn = pl.kernel(
      _sc_body,
      out_type=[part, part, part, part, part],
      mesh=mesh,
      compiler_params=pltpu.CompilerParams(needs_layout_passes=False),
      scratch_types=[
          pltpu.VMEM((NPACK,), jnp.int32),
          pltpu.VMEM((CHUNK,), jnp.int32),
          pltpu.VMEM((CHUNK * L,), f32),
          pltpu.VMEM((CHUNK,), jnp.int32),
          pltpu.VMEM((B * L,), f32),
          pltpu.VMEM((B * L,), f32),
          pltpu.VMEM((B * L,), f32),
          pltpu.VMEM((B * L,), f32),
          pltpu.VMEM((B * L,), f32),
      ],
  )
  return fn(src, attr_flat, batch_packed)


def _finale_body(sum_ref, sq_ref, max_ref, min_ref, cnt_ref,
                 w0, w1, w2, w3, b_ref, out_ref):
  sums = jnp.sum(sum_ref[...], axis=0)
  sqs = jnp.sum(sq_ref[...], axis=0)
  mx = jnp.max(max_ref[...], axis=0)
  mn = jnp.min(min_ref[...], axis=0)
  cnt = jnp.sum(cnt_ref[...], axis=0)
  denom = jnp.maximum(cnt, 1.0)
  m = sums / denom
  var = sqs / denom - m * m

  def dot(x, w):
    return lax.dot_general(x, w[...], (((1,), (1,)), ((), ())),
                           preferred_element_type=jnp.float32)

  out_ref[...] = dot(m, w0) + dot(mn, w1) + dot(mx, w2) + dot(var, w3) + b_ref[...]


def _finale(psum, psq, pmax, pmin, pcnt, W, b):
  w0 = W[:, 0:D]
  w1 = W[:, D:2 * D]
  w2 = W[:, 2 * D:3 * D]
  w3 = W[:, 3 * D:4 * D]
  return pl.pallas_call(
      _finale_body,
      out_shape=jax.ShapeDtypeStruct((B, DY), jnp.float32),
  )(psum, psq, pmax, pmin, pcnt, w0, w1, w2, w3, b.reshape(1, DY))


def kernel(edge_index, edge_attr, batch, W, b):
  src = edge_index[0]
  b4 = batch.astype(jnp.uint32).reshape(NPACK, 4)
  batch_packed = (b4[:, 0] | (b4[:, 1] << 8) | (b4[:, 2] << 16)
                  | (b4[:, 3] << 24)).astype(jnp.int32)
  psum, psq, pmax, pmin, pcnt = _sc_partials(
      src, edge_attr.reshape(E * D), batch_packed)
  rs = lambda p: p.reshape(NW, B, L)
  return _finale(rs(psum), rs(psq), rs(pmax), rs(pmin), rs(pcnt), W, b)


# speedup vs baseline: 38.1446x; 1.1722x over previous
import jax
import jax.numpy as jnp
from jax import lax
from jax.experimental import pallas as pl
from jax.experimental.pallas import tpu as pltpu
from jax.experimental.pallas import tpu_sc as plsc

B = 256
N = 100000
E = 3200000
D = 16
DY = 64

NC = 2
NS = 16
NW = NC * NS
L = 16

EW = E // NW
CHUNK = 400
NCHUNKS = EW // CHUNK
NPACK = N // 4
NBANK = 8


def _sc_body(src_hbm, attr_hbm, batch_hbm,
             out_sum, out_sq, out_max, out_min, out_cnt,
             *scratch):
  (batch_v, idx0, idx1, attr0, attr1,
   acc_sum, acc_sq, acc_cnt) = scratch[:8]
  maxs = scratch[8:8 + NBANK]
  mins = scratch[8 + NBANK:8 + 2 * NBANK]
  si0, sa0, si1, sa1 = scratch[8 + 2 * NBANK:]

  c = lax.axis_index("c")
  s = lax.axis_index("s")
  wid = s * NC + c

  pltpu.sync_copy(batch_hbm, batch_v)

  zeros = jnp.zeros((L,), jnp.float32)
  ninf = jnp.full((L,), -jnp.inf, jnp.float32)
  pinf = jnp.full((L,), jnp.inf, jnp.float32)
  ones = jnp.ones((L,), jnp.float32)

  @pl.loop(0, B)
  def _init(r):
    row = pl.ds(r * L, L)
    acc_sum[row] = zeros
    acc_sq[row] = zeros
    for k in range(NBANK):
      maxs[k][row] = ninf
      mins[k][row] = pinf

  @pl.loop(0, B // L)
  def _initc(r):
    acc_cnt[pl.ds(r * L, L)] = zeros

  base0 = wid * EW

  def start(ci, idx_b, attr_b, sem_i, sem_a):
    base = base0 + ci * CHUNK
    pltpu.async_copy(src_hbm.at[pl.ds(base, CHUNK)], idx_b, sem_i)
    pltpu.async_copy(attr_hbm.at[pl.ds(base * D, CHUNK * D)], attr_b, sem_a)

  def wait(idx_b, attr_b, sem_i, sem_a):
    pltpu.make_async_copy(src_hbm.at[pl.ds(0, CHUNK)], idx_b, sem_i).wait()
    pltpu.make_async_copy(attr_hbm.at[pl.ds(0, CHUNK * D)], attr_b,
                          sem_a).wait()

  def process(idx_b, attr_b):
    @pl.loop(0, CHUNK // L)
    def _group(g):
      iv = idx_b[pl.ds(g * L, L)]
      w = plsc.load_gather(batch_v, [lax.shift_right_logical(iv, 2)])
      sh = lax.shift_left(jnp.bitwise_and(iv, 3), 3)
      segs = jnp.bitwise_and(lax.shift_right_logical(w, sh), 255)
      plsc.addupdate_scatter(acc_cnt, [segs], ones)
      rows = lax.shift_left(segs, 4)
      for j in range(L):
        row = pl.ds(rows[j], L)
        v = attr_b[pl.ds((g * L + j) * L, L)]
        plsc.addupdate(acc_sum.at[row], v)
        plsc.addupdate(acc_sq.at[row], v * v)
        amax = maxs[j % NBANK]
        amin = mins[j % NBANK]
        amax[row] = jnp.maximum(amax[row], v)
        amin[row] = jnp.minimum(amin[row], v)

  start(0, idx0, attr0, si0, sa0)

  @pl.loop(0, NCHUNKS, step=2)
  def _chunk(ci):
    start(ci + 1, idx1, attr1, si1, sa1)
    wait(idx0, attr0, si0, sa0)
    process(idx0, attr0)

    @pl.when(ci + 2 < NCHUNKS)
    def _():
      start(ci + 2, idx0, attr0, si0, sa0)

    wait(idx1, attr1, si1, sa1)
    process(idx1, attr1)

  @pl.loop(0, B)
  def _merge(r):
    row = pl.ds(r * L, L)
    m = maxs[0][row]
    n = mins[0][row]
    for k in range(1, NBANK):
      m = jnp.maximum(m, maxs[k][row])
      n = jnp.minimum(n, mins[k][row])
    maxs[0][row] = m
    mins[0][row] = n

  pltpu.sync_copy(acc_sum, out_sum.at[wid])
  pltpu.sync_copy(acc_sq, out_sq.at[wid])
  pltpu.sync_copy(maxs[0], out_max.at[wid])
  pltpu.sync_copy(mins[0], out_min.at[wid])
  pltpu.sync_copy(acc_cnt, out_cnt.at[wid])


def _sc_partials(src, attr_flat, batch_packed):
  mesh = plsc.VectorSubcoreMesh(
      core_axis_name="c", subcore_axis_name="s",
      num_cores=NC, num_subcores=NS)
  f32 = jnp.float32
  part = jax.ShapeDtypeStruct((NW, B * L), f32)
  cnt = jax.ShapeDtypeStruct((NW, B), f32)
  fn = pl.kernel(
      _sc_body,
      out_type=[part, part, part, part, cnt],
      mesh=mesh,
      compiler_params=pltpu.CompilerParams(needs_layout_passes=False),
      scratch_types=[
          pltpu.VMEM((NPACK,), jnp.int32),
          pltpu.VMEM((CHUNK,), jnp.int32),
          pltpu.VMEM((CHUNK,), jnp.int32),
          pltpu.VMEM((CHUNK * L,), f32),
          pltpu.VMEM((CHUNK * L,), f32),
          pltpu.VMEM((B * L,), f32),
          pltpu.VMEM((B * L,), f32),
          pltpu.VMEM((B,), f32),
      ]
      + [pltpu.VMEM((B * L,), f32) for _ in range(2 * NBANK)]
      + [pltpu.SemaphoreType.DMA] * 4,
  )
  return fn(src, attr_flat, batch_packed)


def _finale_body(sum_ref, sq_ref, max_ref, min_ref, cnt_ref,
                 w0, w1, w2, w3, b_ref, out_ref):
  sums = jnp.sum(sum_ref[...], axis=0)
  sqs = jnp.sum(sq_ref[...], axis=0)
  mx = jnp.max(max_ref[...], axis=0)
  mn = jnp.min(min_ref[...], axis=0)
  cnt = jnp.sum(cnt_ref[...], axis=0)
  denom = jnp.maximum(cnt, 1.0)[:, None]
  m = sums / denom
  var = sqs / denom - m * m

  def dot(x, w):
    return lax.dot_general(x, w[...], (((1,), (1,)), ((), ())),
                           preferred_element_type=jnp.float32)

  out_ref[...] = dot(m, w0) + dot(mn, w1) + dot(mx, w2) + dot(var, w3) + b_ref[...]


def _finale(psum, psq, pmax, pmin, pcnt, W, b):
  w0 = W[:, 0:D]
  w1 = W[:, D:2 * D]
  w2 = W[:, 2 * D:3 * D]
  w3 = W[:, 3 * D:4 * D]
  return pl.pallas_call(
      _finale_body,
      out_shape=jax.ShapeDtypeStruct((B, DY), jnp.float32),
  )(psum, psq, pmax, pmin, pcnt, w0, w1, w2, w3, b.reshape(1, DY))


def kernel(edge_index, edge_attr, batch, W, b):
  src = edge_index[0]
  b4 = batch.astype(jnp.uint32).reshape(NPACK, 4)
  batch_packed = (b4[:, 0] | (b4[:, 1] << 8) | (b4[:, 2] << 16)
                  | (b4[:, 3] << 24)).astype(jnp.int32)
  psum, psq, pmax, pmin, pcnt = _sc_partials(
      src, edge_attr.reshape(E * D), batch_packed)
  rs = lambda p: p.reshape(NW, B, L)
  return _finale(rs(psum), rs(psq), rs(pmax), rs(pmin), pcnt, W, b)
